# Initial kernel scaffold; baseline (speedup 1.0000x reference)
import jax
import jax.numpy as jnp
from jax.experimental import pallas as pl


def kernel(image, boxes, box_ind):
    raise NotImplementedError("write your pallas kernel here")


# speedup vs baseline: 1.4741x; 1.4741x over previous
import functools

import jax
import jax.numpy as jnp
from jax import lax
from jax.experimental import pallas as pl
from jax.experimental.pallas import tpu as pltpu
from jax.experimental.pallas import tpu_sc as plsc

B, C, H, W = 4, 96, 224, 224
CP = 128
CROP_H, CROP_W = 14, 14
P = CROP_H * CROP_W
P_OUT = 200
NB = 1024
NCORES, NSUB = 2, 16
NW = NCORES * NSUB
BPW = NB // NW
NCHUNK = 7
ROWS = 128
L = 16

_F = jnp.float32
_I = jnp.int32

_CPX = [32] * 6 + [16]
_CBASE = [32 * c for c in range(NCHUNK)]


def _iota():
    return lax.iota(_I, L)


def _bc(x, dtype=_I):
    return jnp.full((L,), x, dtype)


def _bget(vec, idx):
    return vec.at[idx].get(mode="promise_in_bounds")


def _sc_body(img_hbm, boxes_hbm, bind_hbm, out_hbm,
             boxes_v, bind_v, idx_buf, rows_buf, out_buf, gsem, osem):
    wid = lax.axis_index("s") * NCORES + lax.axis_index("c")
    base_box = wid * BPW
    pltpu.sync_copy(boxes_hbm.at[pl.ds(base_box * 4, BPW * 4)],
                    boxes_v.at[pl.ds(0, BPW * 4)])
    pltpu.sync_copy(bind_hbm.at[pl.ds(base_box, BPW)],
                    bind_v.at[pl.ds(0, BPW)])

    iota = _iota()
    iota_f = iota.astype(_F)

    def fire(c, slot):
        nr = 4 * _CPX[c]
        idx = idx_buf.at[pl.ds(c * ROWS, nr)]
        dst = rows_buf.at[pl.ds(slot * ROWS, nr)]
        pltpu.async_copy(img_hbm.at[idx], dst, gsem.at[slot])

    def wait_gather(c, slot):
        nr = 4 * _CPX[c]
        idx = idx_buf.at[pl.ds(c * ROWS, nr)]
        dst = rows_buf.at[pl.ds(slot * ROWS, nr)]
        pltpu.make_async_copy(img_hbm.at[idx], dst, gsem.at[slot]).wait()

    def out_dma_refs(box, slot):
        return out_buf.at[slot], out_hbm.at[pl.ds(box * P_OUT, P_OUT)]

    def process_box(n, oslot):
        box = base_box + n
        @pl.when(n >= 2)
        def _():
            src, dst = out_dma_refs(box - 2, oslot)
            pltpu.make_async_copy(src, dst, osem.at[oslot]).wait()

        bx = boxes_v[pl.ds(n * 4, L)]
        y1 = _bget(bx, _bc(0))
        x1 = _bget(bx, _bc(1))
        y2 = _bget(bx, _bc(2))
        x2 = _bget(bx, _bc(3))
        bi16 = bind_v[pl.ds(n, L)]
        bv = _bget(bi16, _bc(0))

        hs = (y2 - y1) * (H - 1.0) / (CROP_H - 1.0)
        ws = (x2 - x1) * (W - 1.0) / (CROP_W - 1.0)
        in_y = y1 * (H - 1.0) + iota_f * hs
        in_x = x1 * (W - 1.0) + iota_f * ws

        vy = jnp.where((in_y >= 0.0) & (in_y <= H - 1.0), 1.0, 0.0)
        vx = jnp.where((in_x >= 0.0) & (in_x <= W - 1.0), 1.0, 0.0)
        ty = in_y.astype(_I)
        tyf = ty.astype(_F)
        yl = in_y - tyf
        ti = jnp.clip(ty, 0, H - 1)
        bb = jnp.clip(jnp.where(in_y > tyf, ty + 1, ty), 0, H - 1)
        tx = in_x.astype(_I)
        txf = tx.astype(_F)
        xl = in_x - txf
        li = jnp.clip(tx, 0, W - 1)
        ri = jnp.clip(jnp.where(in_x > txf, tx + 1, tx), 0, W - 1)

        rt = bv * (H * W) + ti * W
        rb = bv * (H * W) + bb * W

        for c in range(NCHUNK):
            cpx = _CPX[c]
            for h in range(cpx // L):
                p16 = _CBASE[c] + h * L + iota
                pc = jnp.minimum(p16, P - 1)
                ii = lax.div(pc, CROP_W)
                jj = pc - ii * CROP_W
                rtv = _bget(rt, ii)
                rbv = _bget(rb, ii)
                liv = _bget(li, jj)
                riv = _bget(ri, jj)
                o0 = c * ROWS + h * L
                idx_buf[pl.ds(o0 + 0 * cpx, L)] = rtv + liv
                idx_buf[pl.ds(o0 + 1 * cpx, L)] = rtv + riv
                idx_buf[pl.ds(o0 + 2 * cpx, L)] = rbv + liv
                idx_buf[pl.ds(o0 + 3 * cpx, L)] = rbv + riv

        fire(0, 0)
        fire(1, 1)

        for c in range(NCHUNK):
            slot = c % 2
            cpx = _CPX[c]
            wait_gather(c, slot)

            def pixel_body(pli, _, c=c, slot=slot, cpx=cpx):
                p = _CBASE[c] + pli
                i_s = lax.div(p, CROP_W)
                j_s = p - i_s * CROP_W
                xwb = _bget(xl, _bc(j_s))
                ywb = _bget(yl, _bc(i_s))
                vvb = _bget(vy, _bc(i_s)) * _bget(vx, _bc(j_s))
                r0 = slot * ROWS + pli

                def emit():
                    for k in range(C // L):
                        cs = pl.ds(k * L, L)
                        tl = rows_buf[r0 + 0 * cpx, cs]
                        tr = rows_buf[r0 + 1 * cpx, cs]
                        bl = rows_buf[r0 + 2 * cpx, cs]
                        br = rows_buf[r0 + 3 * cpx, cs]
                        top = tl + (tr - tl) * xwb
                        bot = bl + (br - bl) * xwb
                        o = (top + (bot - top) * ywb) * vvb
                        out_buf[oslot, p, cs] = o

                if c == NCHUNK - 1:
                    @pl.when(p < P)
                    def _():
                        emit()
                else:
                    emit()
                return None

            lax.fori_loop(0, cpx, pixel_body, None)

            if c + 2 < NCHUNK:
                fire(c + 2, slot)

        src, dst = out_dma_refs(box, oslot)
        pltpu.async_copy(src, dst, osem.at[oslot])

    def box_pair(bp, _):
        process_box(bp * 2, 0)
        process_box(bp * 2 + 1, 1)
        return None

    lax.fori_loop(0, BPW // 2, box_pair, None)

    for slot, off in ((0, 2), (1, 1)):
        src, dst = out_dma_refs(base_box + BPW - off, slot)
        pltpu.make_async_copy(src, dst, osem.at[slot]).wait()


@jax.jit
def kernel(image, boxes, box_ind):
    img_t = jnp.transpose(image, (0, 2, 3, 1)).reshape(B * H * W, C)
    img_t = jnp.pad(img_t, ((0, 0), (0, CP - C)))
    mesh = plsc.VectorSubcoreMesh(core_axis_name="c", subcore_axis_name="s")
    kfn = pl.kernel(
        _sc_body,
        out_type=jax.ShapeDtypeStruct((NB * P_OUT, C), jnp.float32),
        mesh=mesh,
        scratch_types=[
            pltpu.VMEM((BPW * 4 + L,), jnp.float32),
            pltpu.VMEM((BPW + L,), jnp.int32),
            pltpu.VMEM((NCHUNK * ROWS,), jnp.int32),
            pltpu.VMEM((2 * ROWS, CP), jnp.float32),
            pltpu.VMEM((2, P_OUT, C), jnp.float32),
            pltpu.SemaphoreType.DMA((2,)),
            pltpu.SemaphoreType.DMA((2,)),
        ],
    )
    out_flat = kfn(img_t, boxes.reshape(NB * 4), box_ind)
    out = out_flat.reshape(NB, P_OUT, C)[:, :P].reshape(NB, CROP_H, CROP_W, C)
    return jnp.transpose(out, (0, 3, 1, 2))


# speedup vs baseline: 2.1524x; 1.4601x over previous
import functools

import jax
import jax.numpy as jnp
from jax import lax
from jax.experimental import pallas as pl
from jax.experimental.pallas import tpu as pltpu
from jax.experimental.pallas import tpu_sc as plsc

B, C, H, W = 4, 96, 224, 224
CP = 128
CROP_H, CROP_W = 14, 14
P = CROP_H * CROP_W
P_OUT = 200
NB = 1024
NCORES, NSUB = 2, 16
NW = NCORES * NSUB
BPW = NB // NW
NCHUNK = 7
ROWS = 128
L = 16

_F = jnp.float32
_I = jnp.int32

_CPX = [32] * 6 + [16]
_CBASE = [32 * c for c in range(NCHUNK)]


def _iota():
    return lax.iota(_I, L)


def _bc(x, dtype=_I):
    return jnp.full((L,), x, dtype)


def _bget(vec, idx):
    return vec.at[idx].get(mode="promise_in_bounds")


def _sc_body(img_hbm, boxes_hbm, bind_hbm, out_hbm,
             boxes_v, bind_v, idx_buf, rows_buf, out_buf, gsem, osem):
    wid = lax.axis_index("s") * NCORES + lax.axis_index("c")
    base_box = wid * BPW
    pltpu.sync_copy(boxes_hbm.at[pl.ds(base_box * 4, BPW * 4)],
                    boxes_v.at[pl.ds(0, BPW * 4)])
    pltpu.sync_copy(bind_hbm.at[pl.ds(base_box, BPW)],
                    bind_v.at[pl.ds(0, BPW)])

    iota = _iota()
    iota_f = iota.astype(_F)

    def fire(c, slot):
        nr = 4 * _CPX[c]
        idx = idx_buf.at[pl.ds(c * ROWS, nr)]
        dst = rows_buf.at[pl.ds(slot * ROWS, nr)]
        pltpu.async_copy(img_hbm.at[idx], dst, gsem.at[slot])

    def wait_gather(c, slot):
        nr = 4 * _CPX[c]
        idx = idx_buf.at[pl.ds(c * ROWS, nr)]
        dst = rows_buf.at[pl.ds(slot * ROWS, nr)]
        pltpu.make_async_copy(img_hbm.at[idx], dst, gsem.at[slot]).wait()

    def out_dma_refs(box, slot):
        return out_buf.at[slot], out_hbm.at[pl.ds(box * P_OUT, P_OUT)]

    def process_box(n, oslot):
        box = base_box + n
        @pl.when(n >= 2)
        def _():
            src, dst = out_dma_refs(box - 2, oslot)
            pltpu.make_async_copy(src, dst, osem.at[oslot]).wait()

        bx = boxes_v[pl.ds(n * 4, L)]
        y1 = _bget(bx, _bc(0))
        x1 = _bget(bx, _bc(1))
        y2 = _bget(bx, _bc(2))
        x2 = _bget(bx, _bc(3))
        bi16 = bind_v[pl.ds(n, L)]
        bv = _bget(bi16, _bc(0))

        hs = (y2 - y1) * (H - 1.0) / (CROP_H - 1.0)
        ws = (x2 - x1) * (W - 1.0) / (CROP_W - 1.0)
        in_y = y1 * (H - 1.0) + iota_f * hs
        in_x = x1 * (W - 1.0) + iota_f * ws

        vy = jnp.where((in_y >= 0.0) & (in_y <= H - 1.0), 1.0, 0.0)
        vx = jnp.where((in_x >= 0.0) & (in_x <= W - 1.0), 1.0, 0.0)
        ty = in_y.astype(_I)
        tyf = ty.astype(_F)
        yl = in_y - tyf
        ti = jnp.clip(ty, 0, H - 1)
        bb = jnp.clip(jnp.where(in_y > tyf, ty + 1, ty), 0, H - 1)
        tx = in_x.astype(_I)
        txf = tx.astype(_F)
        xl = in_x - txf
        li = jnp.clip(tx, 0, W - 1)
        ri = jnp.clip(jnp.where(in_x > txf, tx + 1, tx), 0, W - 1)

        rt = bv * (H * W) + ti * W
        rb = bv * (H * W) + bb * W

        for c in range(NCHUNK):
            cpx = _CPX[c]
            for h in range(cpx // L):
                p16 = _CBASE[c] + h * L + iota
                pc = jnp.minimum(p16, P - 1)
                ii = lax.div(pc, CROP_W)
                jj = pc - ii * CROP_W
                rtv = _bget(rt, ii)
                rbv = _bget(rb, ii)
                liv = _bget(li, jj)
                riv = _bget(ri, jj)
                o0 = c * ROWS + h * L
                idx_buf[pl.ds(o0 + 0 * cpx, L)] = rtv + liv
                idx_buf[pl.ds(o0 + 1 * cpx, L)] = rtv + riv
                idx_buf[pl.ds(o0 + 2 * cpx, L)] = rbv + liv
                idx_buf[pl.ds(o0 + 3 * cpx, L)] = rbv + riv

        fire(0, 0)
        fire(1, 1)

        for c in range(NCHUNK):
            slot = c % 2
            cpx = _CPX[c]
            wait_gather(c, slot)

            def pixel_body(pli, _, c=c, slot=slot, cpx=cpx):
                p = _CBASE[c] + pli
                i_s = lax.div(p, CROP_W)
                j_s = p - i_s * CROP_W
                xwb = _bget(xl, _bc(j_s))
                ywb = _bget(yl, _bc(i_s))
                vvb = _bget(vy, _bc(i_s)) * _bget(vx, _bc(j_s))
                r0 = slot * ROWS + pli

                def emit():
                    for k in range(C // L):
                        cs = pl.ds(k * L, L)
                        tl = rows_buf[r0 + 0 * cpx, cs]
                        tr = rows_buf[r0 + 1 * cpx, cs]
                        bl = rows_buf[r0 + 2 * cpx, cs]
                        br = rows_buf[r0 + 3 * cpx, cs]
                        top = tl + (tr - tl) * xwb
                        bot = bl + (br - bl) * xwb
                        o = (top + (bot - top) * ywb) * vvb
                        out_buf[oslot, p, cs] = o

                if c == NCHUNK - 1:
                    @pl.when(p < P)
                    def _():
                        emit()
                else:
                    emit()
                return None

            lax.fori_loop(0, cpx, pixel_body, None)

            if c + 2 < NCHUNK:
                fire(c + 2, slot)

        src, dst = out_dma_refs(box, oslot)
        pltpu.async_copy(src, dst, osem.at[oslot])

    def box_pair(bp, _):
        process_box(bp * 2, 0)
        process_box(bp * 2 + 1, 1)
        return None

    lax.fori_loop(0, BPW // 2, box_pair, None)

    for slot, off in ((0, 2), (1, 1)):
        src, dst = out_dma_refs(base_box + BPW - off, slot)
        pltpu.make_async_copy(src, dst, osem.at[slot]).wait()


def _tc_nhwc_body(img_ref, out_ref):
    eye = (lax.broadcasted_iota(_I, (C, CP), 0)
           == lax.broadcasted_iota(_I, (C, CP), 1)).astype(_F)
    for y in range(_YB):
        xs = img_ref[0, :, y, :]
        out_ref[pl.ds(y * W, W), :] = lax.dot_general(
            xs, eye, (((0,), (0,)), ((), ())),
            preferred_element_type=_F)


_YB = 8


def _tc_nhwc(image):
    grid = (B, H // _YB)
    return pl.pallas_call(
        _tc_nhwc_body,
        grid=grid,
        in_specs=[pl.BlockSpec((1, C, _YB, W), lambda b, t: (b, 0, t, 0))],
        out_specs=pl.BlockSpec((_YB * W, CP), lambda b, t: (b * (H // _YB) + t, 0)),
        out_shape=jax.ShapeDtypeStruct((B * H * W, CP), jnp.float32),
    )(image)


@jax.jit
def kernel(image, boxes, box_ind):
    img_t = _tc_nhwc(image)
    mesh = plsc.VectorSubcoreMesh(core_axis_name="c", subcore_axis_name="s")
    kfn = pl.kernel(
        _sc_body,
        out_type=jax.ShapeDtypeStruct((NB * P_OUT, C), jnp.float32),
        mesh=mesh,
        scratch_types=[
            pltpu.VMEM((BPW * 4 + L,), jnp.float32),
            pltpu.VMEM((BPW + L,), jnp.int32),
            pltpu.VMEM((NCHUNK * ROWS,), jnp.int32),
            pltpu.VMEM((2 * ROWS, CP), jnp.float32),
            pltpu.VMEM((2, P_OUT, C), jnp.float32),
            pltpu.SemaphoreType.DMA((2,)),
            pltpu.SemaphoreType.DMA((2,)),
        ],
    )
    out_flat = kfn(img_t, boxes.reshape(NB * 4), box_ind)
    out = out_flat.reshape(NB, P_OUT, C)[:, :P].reshape(NB, CROP_H, CROP_W, C)
    return jnp.transpose(out, (0, 3, 1, 2))
